# baseline (speedup 1.0000x reference)
import functools

import jax
import jax.numpy as jnp
from jax import lax
from jax.experimental import pallas as pl
from jax.experimental.pallas import tpu as pltpu
from jax.experimental.pallas import tpu_sc as plsc

B, T = 4096, 200
D = 2
N = B * T

_info = plsc.get_sparse_core_info()
NC, NS, L = _info.num_cores, _info.num_subcores, _info.num_lanes
NW = NC * NS
PER_W = N // NW
CHUNK = 6400
NCHUNK = PER_W // CHUNK
GROUPS = CHUNK // L


def _gather_body(idx_hbm, table_hbm, out_hbm, idx_v, q_v, rows_v, out_v, sem):
    wid = lax.axis_index("s") * NC + lax.axis_index("c")
    base = wid * PER_W

    def round_(k, carry):
        off = base + k * CHUNK
        pltpu.sync_copy(idx_hbm.at[pl.ds(off, CHUNK)], idx_v)

        def calc_q(i, c):
            v = idx_v[pl.ds(i * L, L)]
            q_v[pl.ds(i * L, L)] = lax.shift_right_logical(v, 2)
            return c

        lax.fori_loop(0, GROUPS, calc_q, 0)
        pltpu.async_copy(table_hbm.at[q_v], rows_v, sem).wait()

        lane = lax.iota(jnp.int32, L)

        def select(i, c):
            v = idx_v[pl.ds(i * L, L)]
            row = i * L + lane
            col = (v & 3) * 2
            v0 = plsc.load_gather(rows_v, [row, col])
            v1 = plsc.load_gather(rows_v, [row, col + 1])
            oidx = lax.mul(row, 2)
            plsc.store_scatter(out_v, [oidx], v0)
            plsc.store_scatter(out_v, [oidx + 1], v1)
            return c

        lax.fori_loop(0, GROUPS, select, 0)
        pltpu.sync_copy(out_v, out_hbm.at[pl.ds(2 * off, 2 * CHUNK)])
        return carry

    lax.fori_loop(0, NCHUNK, round_, 0)


@jax.jit
def _gather(ids_flat, table8):
    mesh = plsc.VectorSubcoreMesh(core_axis_name="c", subcore_axis_name="s")
    run = pl.kernel(
        _gather_body,
        out_type=jax.ShapeDtypeStruct((N * D,), jnp.float32),
        mesh=mesh,
        scratch_types=[
            pltpu.VMEM((CHUNK,), jnp.int32),
            pltpu.VMEM((CHUNK,), jnp.int32),
            pltpu.VMEM((CHUNK, 8), jnp.float32),
            pltpu.VMEM((CHUNK * D,), jnp.float32),
            pltpu.SemaphoreType.DMA,
        ],
        compiler_params=pltpu.CompilerParams(
            use_tc_tiling_on_sc=False, needs_layout_passes=False
        ),
    )
    return run(ids_flat, table8)


def kernel(sensor_ids, positions):
    ids_flat = sensor_ids.reshape(N).astype(jnp.int32)
    table8 = positions.reshape(250000, 8)
    out = _gather(ids_flat, table8)
    return out.reshape(B, T, D)


# speedup vs baseline: 13.6338x; 13.6338x over previous
import functools

import jax
import jax.numpy as jnp
from jax import lax
from jax.experimental import pallas as pl
from jax.experimental.pallas import tpu as pltpu
from jax.experimental.pallas import tpu_sc as plsc

B, T = 4096, 200
D = 2
N = B * T
V = 1000000
VMAIN = 999936
MAIN_ROWS = 249984

_info = plsc.get_sparse_core_info()
NC, NS, L = _info.num_cores, _info.num_subcores, _info.num_lanes
NW = NC * NS
PER_W = N // NW
C = 2560
NCH = PER_W // C
GR = C // L


def _gather_body(idx_hbm, table_hbm, tail_hbm, out_hbm,
                 idx_v, sx_v, sy_v, col_v, rowsx_v, rowsy_v, tail_v, out_v,
                 semx, semy):
    wid = lax.axis_index("s") * NC + lax.axis_index("c")
    base = wid * PER_W
    pltpu.sync_copy(idx_hbm.at[pl.ds(base, PER_W)], idx_v)
    pltpu.sync_copy(tail_hbm, tail_v)
    lane = lax.iota(jnp.int32, L)

    def chunk(k, carry):
        m0 = k * C

        def prep(i, c):
            mv = lane + (m0 + i * L)
            n = (mv & 127) * 200 + lax.shift_right_logical(mv, 7)
            v = plsc.load_gather(idx_v, [n])
            is_t = v >= VMAIN
            sx = lax.shift_right_logical(v, 3) + lax.shift_left(
                lax.shift_right_logical(v, 7), 4)
            sx = jnp.where(is_t, 0, sx)
            col = jnp.where(is_t, 1024 + (v - VMAIN), v & 7)
            sx_v[pl.ds(i * L, L)] = sx
            sy_v[pl.ds(i * L, L)] = sx + 16
            col_v[pl.ds(i * L, L)] = col
            return c

        lax.fori_loop(0, GR, prep, 0)
        cx = pltpu.async_copy(table_hbm.at[sx_v], rowsx_v, semx)
        cy = pltpu.async_copy(table_hbm.at[sy_v], rowsy_v, semy)
        cx.wait()
        cy.wait()

        def sel(i, c):
            m = m0 + i * L
            rl = lane + i * L
            cf = col_v[pl.ds(i * L, L)]
            is_t = cf >= 1024
            csafe = cf & 7
            vx = plsc.load_gather(rowsx_v, [rl, csafe])
            vy = plsc.load_gather(rowsy_v, [rl, csafe])
            tj = jnp.where(is_t, cf - 1024, 0)
            tx = plsc.load_gather(tail_v, [tj])
            ty = plsc.load_gather(tail_v, [tj + 64])
            v0 = jnp.where(is_t, tx, vx)
            v1 = jnp.where(is_t, ty, vy)
            tq = lax.shift_right_logical(m, 7)
            cs = m & 127
            out_v[tq, pl.ds(cs, L)] = v0
            out_v[tq, pl.ds(cs + 128, L)] = v1
            return c

        lax.fori_loop(0, GR, sel, 0)
        return carry

    lax.fori_loop(0, NCH, chunk, 0)
    pltpu.sync_copy(out_v, out_hbm.at[:, wid])


@jax.jit
def _gather(ids_flat, table8, tail_flat):
    mesh = plsc.VectorSubcoreMesh(core_axis_name="c", subcore_axis_name="s")
    run = pl.kernel(
        _gather_body,
        out_type=jax.ShapeDtypeStruct((T, NW, 2 * 128), jnp.float32),
        mesh=mesh,
        scratch_types=[
            pltpu.VMEM((PER_W,), jnp.int32),
            pltpu.VMEM((C,), jnp.int32),
            pltpu.VMEM((C,), jnp.int32),
            pltpu.VMEM((C,), jnp.int32),
            pltpu.VMEM((C, 8), jnp.float32),
            pltpu.VMEM((C, 8), jnp.float32),
            pltpu.VMEM((128,), jnp.float32),
            pltpu.VMEM((T, 2 * 128), jnp.float32),
            pltpu.SemaphoreType.DMA,
            pltpu.SemaphoreType.DMA,
        ],
        compiler_params=pltpu.CompilerParams(
            use_tc_tiling_on_sc=False, needs_layout_passes=False
        ),
    )
    return run(ids_flat, table8, tail_flat)


def kernel(sensor_ids, positions):
    ids_flat = sensor_ids.reshape(N).astype(jnp.int32)
    main = positions[:VMAIN]
    table8 = (
        main.reshape(VMAIN // 128, 128, 2)
        .transpose(0, 2, 1)
        .reshape(MAIN_ROWS, 8)
    )
    tail = positions[VMAIN:]
    tail_flat = jnp.concatenate([tail[:, 0], tail[:, 1]])
    out3 = _gather(ids_flat, table8, tail_flat)
    return (
        out3.reshape(T, NW, 2, 128)
        .transpose(1, 3, 0, 2)
        .reshape(B, T, D)
    )


# speedup vs baseline: 17.0874x; 1.2533x over previous
import functools

import jax
import jax.numpy as jnp
from jax import lax
from jax.experimental import pallas as pl
from jax.experimental.pallas import tpu as pltpu
from jax.experimental.pallas import tpu_sc as plsc

B, T = 4096, 200
D = 2
N = B * T
V = 1000000
VMAIN = 999936
MAIN_ROWS = 249984

_info = plsc.get_sparse_core_info()
NC, NS, L = _info.num_cores, _info.num_subcores, _info.num_lanes
NW = NC * NS
PER_W = N // NW
C = 1280
NCH = PER_W // C
GR = C // L
UNROLL = 4


def _gather_body(idx_hbm, table_hbm, tail_hbm, out_hbm,
                 idx_v, tail_v, out_v,
                 sx_a, sy_a, col_a, sx_b, sy_b, col_b,
                 rx_a, ry_a, rx_b, ry_b,
                 semx_a, semy_a, semx_b, semy_b):
    wid = lax.axis_index("s") * NC + lax.axis_index("c")
    pltpu.sync_copy(idx_hbm.at[:, wid], idx_v)
    pltpu.sync_copy(tail_hbm, tail_v)
    lane = lax.iota(jnp.int32, L)

    def prep_group(m, sx_v, sy_v, col_v, j):
        t = m // 128
        c0 = m - t * 128
        v = idx_v[t // 8, pl.ds((t % 8) * 128 + c0, L)]
        is_t = v >= VMAIN
        sx = lax.shift_right_logical(v, 3) + lax.shift_left(
            lax.shift_right_logical(v, 7), 4)
        sx = jnp.where(is_t, 0, sx)
        col = jnp.where(is_t, v - (VMAIN - 1024), v & 7)
        sx_v[pl.ds(j, L)] = sx
        sy_v[pl.ds(j, L)] = sx + 16
        col_v[pl.ds(j, L)] = col

    def sel_group(m, col_v, rx_v, ry_v, j):
        rl = lane + j
        cf = col_v[pl.ds(j, L)]
        is_t = cf >= 1024
        csafe = cf & 7
        vx = plsc.load_gather(rx_v, [rl, csafe])
        vy = plsc.load_gather(ry_v, [rl, csafe])
        tj = jnp.where(is_t, cf - 1024, 0)
        tx = plsc.load_gather(tail_v, [tj])
        ty = plsc.load_gather(tail_v, [tj + 64])
        v0 = jnp.where(is_t, tx, vx)
        v1 = jnp.where(is_t, ty, vy)
        tq = m // 128
        cs = m - tq * 128
        out_v[tq, pl.ds(cs, L)] = v0
        out_v[tq, pl.ds(cs + 128, L)] = v1

    def prep(c, sx_v, sy_v, col_v):
        def step(i, carry):
            j0 = i * (L * UNROLL)
            for u in range(UNROLL):
                j = j0 + u * L
                prep_group(c * C + j, sx_v, sy_v, col_v, j)
            return carry
        lax.fori_loop(0, GR // UNROLL, step, 0)

    def select(c, col_v, rx_v, ry_v):
        def step(i, carry):
            j0 = i * (L * UNROLL)
            for u in range(UNROLL):
                j = j0 + u * L
                sel_group(c * C + j, col_v, rx_v, ry_v, j)
            return carry
        lax.fori_loop(0, GR // UNROLL, step, 0)

    def fire(sx_v, sy_v, rx_v, ry_v, semx, semy):
        pltpu.async_copy(table_hbm.at[sx_v], rx_v, semx)
        pltpu.async_copy(table_hbm.at[sy_v], ry_v, semy)

    def drain(sx_v, sy_v, rx_v, ry_v, semx, semy):
        pltpu.make_async_copy(table_hbm.at[sx_v], rx_v, semx).wait()
        pltpu.make_async_copy(table_hbm.at[sy_v], ry_v, semy).wait()

    prep(0, sx_a, sy_a, col_a)
    fire(sx_a, sy_a, rx_a, ry_a, semx_a, semy_a)

    def pipe(k2, carry):
        c0 = 2 * k2
        c1 = c0 + 1
        prep(c1, sx_b, sy_b, col_b)
        drain(sx_a, sy_a, rx_a, ry_a, semx_a, semy_a)
        fire(sx_b, sy_b, rx_b, ry_b, semx_b, semy_b)
        select(c0, col_a, rx_a, ry_a)

        @pl.when(c1 + 1 < NCH)
        def _():
            prep(c1 + 1, sx_a, sy_a, col_a)
            fire(sx_a, sy_a, rx_a, ry_a, semx_a, semy_a)

        drain(sx_b, sy_b, rx_b, ry_b, semx_b, semy_b)
        select(c1, col_b, rx_b, ry_b)
        return carry

    lax.fori_loop(0, NCH // 2, pipe, 0)
    pltpu.sync_copy(out_v, out_hbm.at[:, wid])


@jax.jit
def _gather(ids_native, table8, tail_flat):
    mesh = plsc.VectorSubcoreMesh(core_axis_name="c", subcore_axis_name="s")
    run = pl.kernel(
        _gather_body,
        out_type=jax.ShapeDtypeStruct((T, NW, 2 * 128), jnp.float32),
        mesh=mesh,
        scratch_types=[
            pltpu.VMEM((25, 1024), jnp.int32),
            pltpu.VMEM((128,), jnp.float32),
            pltpu.VMEM((T, 2 * 128), jnp.float32),
            pltpu.VMEM((C,), jnp.int32),
            pltpu.VMEM((C,), jnp.int32),
            pltpu.VMEM((C,), jnp.int32),
            pltpu.VMEM((C,), jnp.int32),
            pltpu.VMEM((C,), jnp.int32),
            pltpu.VMEM((C,), jnp.int32),
            pltpu.VMEM((C, 8), jnp.float32),
            pltpu.VMEM((C, 8), jnp.float32),
            pltpu.VMEM((C, 8), jnp.float32),
            pltpu.VMEM((C, 8), jnp.float32),
            pltpu.SemaphoreType.DMA,
            pltpu.SemaphoreType.DMA,
            pltpu.SemaphoreType.DMA,
            pltpu.SemaphoreType.DMA,
        ],
        compiler_params=pltpu.CompilerParams(
            use_tc_tiling_on_sc=False, needs_layout_passes=False
        ),
    )
    return run(ids_native, table8, tail_flat)


def kernel(sensor_ids, positions):
    ids = sensor_ids.astype(jnp.int32)
    ids_native = (
        ids.reshape(32, 128, 25, 8)
        .transpose(2, 0, 3, 1)
        .reshape(25, 32, 1024)
    )
    main = positions[:VMAIN]
    table8 = (
        main.reshape(VMAIN // 128, 128, 2)
        .transpose(0, 2, 1)
        .reshape(MAIN_ROWS, 8)
    )
    tail = positions[VMAIN:]
    tail_flat = jnp.concatenate([tail[:, 0], tail[:, 1]])
    out3 = _gather(ids_native, table8, tail_flat)
    return (
        out3.reshape(T, NW, 2, 128)
        .transpose(1, 3, 0, 2)
        .reshape(B, T, D)
    )
